# Initial kernel scaffold; baseline (speedup 1.0000x reference)
import functools

import jax
import jax.numpy as jnp
from jax import lax
from jax.experimental import pallas as pl
from jax.experimental.pallas import tpu as pltpu
from jax.experimental.pallas import tpu_sc as plsc

N = 10000
B = 5000
E = 160000
D = 64
EPS = 1e-9

NC, NS = 2, 16
NW = NC * NS
CHUNK = 128
CHUNKS_PER_W = 40
E_PAD = NW * CHUNKS_PER_W * CHUNK
B_PAD = 5120
ROWS_PER_TILE = B_PAD // NS

ROW_BLOCK = 1000
CB_ROWS = 500


def _dense_body(cf_ref, tf_ref, cff_ref, ie_ref,
                cW1, cb1, cW2, cb2, cW3, cb3,
                tW1, tb1, tW2, tb2, tW3, tb3,
                cfW, cfb, Wq, Wk, Wv, out_ref):
    def norm_rows(x):
        n = jnp.sqrt(jnp.sum(x * x, axis=-1, keepdims=True))
        return x / jnp.maximum(n, EPS)

    def mlp(x, W1, b1, W2, b2, W3, b3):
        h = jnp.maximum(
            jnp.dot(x, W1[...], preferred_element_type=jnp.float32) + b1[...], 0.0)
        h = jnp.maximum(
            jnp.dot(h, W2[...], preferred_element_type=jnp.float32) + b2[...], 0.0)
        return jnp.dot(h, W3[...], preferred_element_type=jnp.float32) + b3[...]

    c = mlp(norm_rows(cf_ref[...]), cW1, cb1, cW2, cb2, cW3, cb3)
    t = mlp(norm_rows(tf_ref[...]), tW1, tb1, tW2, tb2, tW3, tb3)
    cfl = jnp.dot(cff_ref[...], cfW[...], preferred_element_type=jnp.float32) + cfb[...]
    feats = [c, t, cfl, ie_ref[...]]
    q = [jnp.dot(f, Wq[...], preferred_element_type=jnp.float32) for f in feats]
    k = [jnp.dot(f, Wk[...], preferred_element_type=jnp.float32) for f in feats]
    v = [jnp.dot(f, Wv[...], preferred_element_type=jnp.float32) for f in feats]
    acc = None
    for m in range(4):
        s = [jnp.sum(q[m] * k[j], axis=-1, keepdims=True) * 0.125 for j in range(4)]
        mx = jnp.maximum(jnp.maximum(s[0], s[1]), jnp.maximum(s[2], s[3]))
        e = [jnp.exp(sj - mx) for sj in s]
        den = e[0] + e[1] + e[2] + e[3]
        o = (e[0] * v[0] + e[1] * v[1] + e[2] * v[2] + e[3] * v[3]) / den
        mu = jnp.mean(o, axis=-1, keepdims=True)
        var = jnp.mean((o - mu) ** 2, axis=-1, keepdims=True)
        o = (o - mu) * lax.rsqrt(var + 1e-5)
        acc = o if acc is None else acc + o
    out_ref[...] = acc * 0.25


def _dense_stage(content, text, cff, ie, weights):
    row = lambda i: (i, 0)
    full = lambda i: (0, 0)
    in_specs = [
        pl.BlockSpec((ROW_BLOCK, 256), row),
        pl.BlockSpec((ROW_BLOCK, 256), row),
        pl.BlockSpec((ROW_BLOCK, D), row),
        pl.BlockSpec((ROW_BLOCK, D), row),
    ] + [pl.BlockSpec(w.shape, full) for w in weights]
    return pl.pallas_call(
        _dense_body,
        grid=(N // ROW_BLOCK,),
        in_specs=in_specs,
        out_specs=pl.BlockSpec((ROW_BLOCK, D), row),
        out_shape=jax.ShapeDtypeStruct((N, D), jnp.float32),
    )(content, text, cff, ie, *weights)


def _sc_stage(fused, iidx, bidx, ones, zagg, zdeg):
    mesh = plsc.VectorSubcoreMesh(core_axis_name="c", subcore_axis_name="s")

    @functools.partial(
        pl.kernel,
        out_type=[
            jax.ShapeDtypeStruct((NC, B_PAD, D), jnp.float32),
            jax.ShapeDtypeStruct((NC, B_PAD, 16), jnp.float32),
        ],
        mesh=mesh,
        scratch_types=[
            pltpu.VMEM((CHUNKS_PER_W, CHUNK), jnp.int32),
            pltpu.VMEM((CHUNKS_PER_W, CHUNK), jnp.int32),
            pltpu.VMEM((CHUNK, D), jnp.float32),
            pltpu.VMEM((CHUNK, 16), jnp.float32),
            pltpu.VMEM_SHARED((B_PAD, D), jnp.float32),
            pltpu.VMEM_SHARED((B_PAD, 16), jnp.float32),
        ],
    )
    def sc_kernel(fused_hbm, iidx_hbm, bidx_hbm, ones_hbm, zagg_hbm, zdeg_hbm,
                  agg_out, deg_out, iidx_v, bidx_v, rows_v, ones_v, agg_s, deg_s):
        c = lax.axis_index("c")
        s = lax.axis_index("s")
        w = s * NC + c
        pltpu.sync_copy(iidx_hbm.at[w], iidx_v)
        pltpu.sync_copy(bidx_hbm.at[w], bidx_v)
        pltpu.sync_copy(ones_hbm, ones_v)
        r0 = s * ROWS_PER_TILE
        pltpu.sync_copy(zagg_hbm.at[pl.ds(r0, ROWS_PER_TILE)],
                        agg_s.at[pl.ds(r0, ROWS_PER_TILE)])
        pltpu.sync_copy(zdeg_hbm.at[pl.ds(r0, ROWS_PER_TILE)],
                        deg_s.at[pl.ds(r0, ROWS_PER_TILE)])
        plsc.subcore_barrier()

        def body(j, carry):
            pltpu.sync_copy(fused_hbm.at[iidx_v.at[j]], rows_v)
            pltpu.sync_copy(rows_v, agg_s.at[bidx_v.at[j]], add=True)
            pltpu.sync_copy(ones_v, deg_s.at[bidx_v.at[j]], add=True)
            return carry

        lax.fori_loop(0, CHUNKS_PER_W, body, 0)
        plsc.subcore_barrier()
        pltpu.sync_copy(agg_s.at[pl.ds(r0, ROWS_PER_TILE)],
                        agg_out.at[c, pl.ds(r0, ROWS_PER_TILE)])
        pltpu.sync_copy(deg_s.at[pl.ds(r0, ROWS_PER_TILE)],
                        deg_out.at[c, pl.ds(r0, ROWS_PER_TILE)])

    return sc_kernel(fused, iidx, bidx, ones, zagg, zdeg)


def _combine_body(a_ref, d_ref, o_ref):
    a = a_ref[0] + a_ref[1]
    dd = d_ref[0][:, :1] + d_ref[1][:, :1]
    o_ref[...] = a / (dd + EPS)


def _combine_stage(agg_p, deg_p):
    return pl.pallas_call(
        _combine_body,
        grid=(B // CB_ROWS,),
        in_specs=[
            pl.BlockSpec((NC, CB_ROWS, D), lambda i: (0, i, 0)),
            pl.BlockSpec((NC, CB_ROWS, 16), lambda i: (0, i, 0)),
        ],
        out_specs=pl.BlockSpec((CB_ROWS, D), lambda i: (i, 0)),
        out_shape=jax.ShapeDtypeStruct((B, D), jnp.float32),
    )(agg_p, deg_p)


def kernel(content_feature, text_feature, cf_feature, item_idx, bundle_idx,
           c_W1, c_b1, c_W2, c_b2, c_W3, c_b3,
           t_W1, t_b1, t_W2, t_b2, t_W3, t_b3,
           cf_W, cf_b, item_emb, Wq, Wk, Wv):
    weights = [
        c_W1, c_b1.reshape(1, -1), c_W2, c_b2.reshape(1, -1),
        c_W3, c_b3.reshape(1, -1),
        t_W1, t_b1.reshape(1, -1), t_W2, t_b2.reshape(1, -1),
        t_W3, t_b3.reshape(1, -1),
        cf_W, cf_b.reshape(1, -1), Wq, Wk, Wv,
    ]
    fused = _dense_stage(content_feature, text_feature, cf_feature, item_emb,
                         weights)

    pad = E_PAD - E
    iidx = jnp.concatenate(
        [item_idx.astype(jnp.int32), jnp.zeros((pad,), jnp.int32)]
    ).reshape(NW, CHUNKS_PER_W, CHUNK)
    bidx = jnp.concatenate(
        [bundle_idx.astype(jnp.int32), jnp.full((pad,), B_PAD - 1, jnp.int32)]
    ).reshape(NW, CHUNKS_PER_W, CHUNK)
    ones = jnp.ones((CHUNK, 16), jnp.float32)
    zagg = jnp.zeros((B_PAD, D), jnp.float32)
    zdeg = jnp.zeros((B_PAD, 16), jnp.float32)

    agg_p, deg_p = _sc_stage(fused, iidx, bidx, ones, zagg, zdeg)
    return _combine_stage(agg_p, deg_p)


# speedup vs baseline: 5.1546x; 5.1546x over previous
import functools

import jax
import jax.numpy as jnp
from jax import lax
from jax.experimental import pallas as pl
from jax.experimental.pallas import tpu as pltpu
from jax.experimental.pallas import tpu_sc as plsc

N = 10000
B = 5000
E = 160000
D = 64
EPS = 1e-9

NC, NS = 2, 16
NW = NC * NS
CHUNK = 128
CHUNKS_PER_W = 40
E_PAD = NW * CHUNKS_PER_W * CHUNK
B_PAD = 5120
ROWS_PER_TILE = B_PAD // NS

ROW_BLOCK = 1000
CB_ROWS = 1000


def _dense_body(cf_ref, tf_ref, cff_ref, ie_ref,
                cW1, cb1, cW2, cb2, cW3, cb3,
                tW1, tb1, tW2, tb2, tW3, tb3,
                cfW, cfb, Wq, Wk, Wv, out_ref):
    def norm_rows(x):
        n = jnp.sqrt(jnp.sum(x * x, axis=-1, keepdims=True))
        return x / jnp.maximum(n, EPS)

    def mlp(x, W1, b1, W2, b2, W3, b3):
        h = jnp.maximum(
            jnp.dot(x, W1[...], preferred_element_type=jnp.float32) + b1[...], 0.0)
        h = jnp.maximum(
            jnp.dot(h, W2[...], preferred_element_type=jnp.float32) + b2[...], 0.0)
        return jnp.dot(h, W3[...], preferred_element_type=jnp.float32) + b3[...]

    c = mlp(norm_rows(cf_ref[...]), cW1, cb1, cW2, cb2, cW3, cb3)
    t = mlp(norm_rows(tf_ref[...]), tW1, tb1, tW2, tb2, tW3, tb3)
    cfl = jnp.dot(cff_ref[...], cfW[...], preferred_element_type=jnp.float32) + cfb[...]
    feats = [c, t, cfl, ie_ref[...]]
    q = [jnp.dot(f, Wq[...], preferred_element_type=jnp.float32) for f in feats]
    k = [jnp.dot(f, Wk[...], preferred_element_type=jnp.float32) for f in feats]
    v = [jnp.dot(f, Wv[...], preferred_element_type=jnp.float32) for f in feats]
    acc = None
    for m in range(4):
        s = [jnp.sum(q[m] * k[j], axis=-1, keepdims=True) * 0.125 for j in range(4)]
        mx = jnp.maximum(jnp.maximum(s[0], s[1]), jnp.maximum(s[2], s[3]))
        e = [jnp.exp(sj - mx) for sj in s]
        den = e[0] + e[1] + e[2] + e[3]
        o = (e[0] * v[0] + e[1] * v[1] + e[2] * v[2] + e[3] * v[3]) / den
        mu = jnp.mean(o, axis=-1, keepdims=True)
        var = jnp.mean((o - mu) ** 2, axis=-1, keepdims=True)
        o = (o - mu) * lax.rsqrt(var + 1e-5)
        acc = o if acc is None else acc + o
    out_ref[...] = acc * 0.25


def _dense_stage(content, text, cff, ie, weights):
    row = lambda i: (i, 0)
    full = lambda i: (0, 0)
    in_specs = [
        pl.BlockSpec((ROW_BLOCK, 256), row),
        pl.BlockSpec((ROW_BLOCK, 256), row),
        pl.BlockSpec((ROW_BLOCK, D), row),
        pl.BlockSpec((ROW_BLOCK, D), row),
    ] + [pl.BlockSpec(w.shape, full) for w in weights]
    return pl.pallas_call(
        _dense_body,
        grid=(N // ROW_BLOCK,),
        in_specs=in_specs,
        out_specs=pl.BlockSpec((ROW_BLOCK, D), row),
        out_shape=jax.ShapeDtypeStruct((N, D), jnp.float32),
    )(content, text, cff, ie, *weights)


def _sc_stage(fused, iidx, bidx, ones, zagg, zdeg):
    mesh = plsc.VectorSubcoreMesh(core_axis_name="c", subcore_axis_name="s")

    @functools.partial(
        pl.kernel,
        out_type=[
            jax.ShapeDtypeStruct((NC, B_PAD, D), jnp.float32),
            jax.ShapeDtypeStruct((NC, B_PAD, 16), jnp.float32),
        ],
        mesh=mesh,
        compiler_params=pltpu.CompilerParams(use_tc_tiling_on_sc=False),
        scratch_types=[
            pltpu.VMEM((CHUNKS_PER_W, CHUNK), jnp.int32),
            pltpu.VMEM((CHUNKS_PER_W, CHUNK), jnp.int32),
            pltpu.VMEM((CHUNK, D), jnp.float32),
            pltpu.VMEM((CHUNK, 16), jnp.float32),
            pltpu.VMEM_SHARED((B_PAD, D), jnp.float32),
            pltpu.VMEM_SHARED((B_PAD, 16), jnp.float32),
        ],
    )
    def sc_kernel(fused_hbm, iidx_hbm, bidx_hbm, ones_hbm, zagg_hbm, zdeg_hbm,
                  agg_out, deg_out, iidx_v, bidx_v, rows_v, ones_v, agg_s, deg_s):
        c = lax.axis_index("c")
        s = lax.axis_index("s")
        w = s * NC + c
        pltpu.sync_copy(iidx_hbm.at[w], iidx_v)
        pltpu.sync_copy(bidx_hbm.at[w], bidx_v)
        pltpu.sync_copy(ones_hbm, ones_v)
        r0 = s * ROWS_PER_TILE
        pltpu.sync_copy(zagg_hbm.at[pl.ds(r0, ROWS_PER_TILE)],
                        agg_s.at[pl.ds(r0, ROWS_PER_TILE)])
        pltpu.sync_copy(zdeg_hbm.at[pl.ds(r0, ROWS_PER_TILE)],
                        deg_s.at[pl.ds(r0, ROWS_PER_TILE)])
        plsc.subcore_barrier()

        def body(j, carry):
            pltpu.sync_copy(fused_hbm.at[iidx_v.at[j]], rows_v)
            pltpu.sync_copy(rows_v, agg_s.at[bidx_v.at[j]], add=True)
            pltpu.sync_copy(ones_v, deg_s.at[bidx_v.at[j]], add=True)
            return carry

        lax.fori_loop(0, CHUNKS_PER_W, body, 0)
        plsc.subcore_barrier()
        pltpu.sync_copy(agg_s.at[pl.ds(r0, ROWS_PER_TILE)],
                        agg_out.at[c, pl.ds(r0, ROWS_PER_TILE)])
        pltpu.sync_copy(deg_s.at[pl.ds(r0, ROWS_PER_TILE)],
                        deg_out.at[c, pl.ds(r0, ROWS_PER_TILE)])

    return sc_kernel(fused, iidx, bidx, ones, zagg, zdeg)


def _combine_body(a_ref, d_ref, o_ref):
    a = a_ref[0] + a_ref[1]
    dd = d_ref[0][:, :1] + d_ref[1][:, :1]
    o_ref[...] = a / (dd + EPS)


def _combine_stage(agg_p, deg_p):
    return pl.pallas_call(
        _combine_body,
        grid=(B // CB_ROWS,),
        in_specs=[
            pl.BlockSpec((NC, CB_ROWS, D), lambda i: (0, i, 0)),
            pl.BlockSpec((NC, CB_ROWS, 16), lambda i: (0, i, 0)),
        ],
        out_specs=pl.BlockSpec((CB_ROWS, D), lambda i: (i, 0)),
        out_shape=jax.ShapeDtypeStruct((B, D), jnp.float32),
    )(agg_p, deg_p)


def kernel(content_feature, text_feature, cf_feature, item_idx, bundle_idx,
           c_W1, c_b1, c_W2, c_b2, c_W3, c_b3,
           t_W1, t_b1, t_W2, t_b2, t_W3, t_b3,
           cf_W, cf_b, item_emb, Wq, Wk, Wv):
    weights = [
        c_W1, c_b1.reshape(1, -1), c_W2, c_b2.reshape(1, -1),
        c_W3, c_b3.reshape(1, -1),
        t_W1, t_b1.reshape(1, -1), t_W2, t_b2.reshape(1, -1),
        t_W3, t_b3.reshape(1, -1),
        cf_W, cf_b.reshape(1, -1), Wq, Wk, Wv,
    ]
    fused = _dense_stage(content_feature, text_feature, cf_feature, item_emb,
                         weights)

    pad = E_PAD - E
    iidx = jnp.concatenate(
        [item_idx.astype(jnp.int32), jnp.zeros((pad,), jnp.int32)]
    ).reshape(NW, CHUNKS_PER_W, CHUNK)
    bidx = jnp.concatenate(
        [bundle_idx.astype(jnp.int32), jnp.full((pad,), B_PAD - 1, jnp.int32)]
    ).reshape(NW, CHUNKS_PER_W, CHUNK)
    ones = jnp.ones((CHUNK, 16), jnp.float32)
    zagg = jnp.zeros((B_PAD, D), jnp.float32)
    zdeg = jnp.zeros((B_PAD, 16), jnp.float32)

    agg_p, deg_p = _sc_stage(fused, iidx, bidx, ones, zagg, zdeg)
    return _combine_stage(agg_p, deg_p)


# speedup vs baseline: 5.4827x; 1.0637x over previous
import functools

import jax
import jax.numpy as jnp
from jax import lax
from jax.experimental import pallas as pl
from jax.experimental.pallas import tpu as pltpu
from jax.experimental.pallas import tpu_sc as plsc

N = 10000
B = 5000
E = 160000
D = 64
EPS = 1e-9

NC, NS = 2, 16
NW = NC * NS
CHUNK = 128
CHUNKS_PER_W = 40
E_PAD = NW * CHUNKS_PER_W * CHUNK
B_PAD = 5120
ROWS_PER_TILE = B_PAD // NS

ROW_BLOCK = 1000
CB_ROWS = 1000


def _dense_body(cf_ref, tf_ref, cff_ref, ie_ref,
                cW1, cb1, cW2, cb2, cW3, cb3,
                tW1, tb1, tW2, tb2, tW3, tb3,
                cfW, cfb, Wq, Wk, Wv, out_ref):
    def norm_rows(x):
        n = jnp.sqrt(jnp.sum(x * x, axis=-1, keepdims=True))
        return x / jnp.maximum(n, EPS)

    def mlp(x, W1, b1, W2, b2, W3, b3):
        h = jnp.maximum(
            jnp.dot(x, W1[...], preferred_element_type=jnp.float32) + b1[...], 0.0)
        h = jnp.maximum(
            jnp.dot(h, W2[...], preferred_element_type=jnp.float32) + b2[...], 0.0)
        return jnp.dot(h, W3[...], preferred_element_type=jnp.float32) + b3[...]

    c = mlp(norm_rows(cf_ref[...]), cW1, cb1, cW2, cb2, cW3, cb3)
    t = mlp(norm_rows(tf_ref[...]), tW1, tb1, tW2, tb2, tW3, tb3)
    cfl = jnp.dot(cff_ref[...], cfW[...], preferred_element_type=jnp.float32) + cfb[...]
    feats = [c, t, cfl, ie_ref[...]]
    q = [jnp.dot(f, Wq[...], preferred_element_type=jnp.float32) for f in feats]
    k = [jnp.dot(f, Wk[...], preferred_element_type=jnp.float32) for f in feats]
    v = [jnp.dot(f, Wv[...], preferred_element_type=jnp.float32) for f in feats]
    acc = None
    for m in range(4):
        s = [jnp.sum(q[m] * k[j], axis=-1, keepdims=True) * 0.125 for j in range(4)]
        mx = jnp.maximum(jnp.maximum(s[0], s[1]), jnp.maximum(s[2], s[3]))
        e = [jnp.exp(sj - mx) for sj in s]
        den = e[0] + e[1] + e[2] + e[3]
        o = (e[0] * v[0] + e[1] * v[1] + e[2] * v[2] + e[3] * v[3]) / den
        mu = jnp.mean(o, axis=-1, keepdims=True)
        var = jnp.mean((o - mu) ** 2, axis=-1, keepdims=True)
        o = (o - mu) * lax.rsqrt(var + 1e-5)
        acc = o if acc is None else acc + o
    out_ref[...] = acc * 0.25


def _dense_stage(content, text, cff, ie, weights):
    row = lambda i: (i, 0)
    full = lambda i: (0, 0)
    in_specs = [
        pl.BlockSpec((ROW_BLOCK, 256), row),
        pl.BlockSpec((ROW_BLOCK, 256), row),
        pl.BlockSpec((ROW_BLOCK, D), row),
        pl.BlockSpec((ROW_BLOCK, D), row),
    ] + [pl.BlockSpec(w.shape, full) for w in weights]
    return pl.pallas_call(
        _dense_body,
        grid=(N // ROW_BLOCK,),
        in_specs=in_specs,
        out_specs=pl.BlockSpec((ROW_BLOCK, D), row),
        out_shape=jax.ShapeDtypeStruct((N, D), jnp.float32),
    )(content, text, cff, ie, *weights)


def _sc_stage(fused, iidx, bidx, ones, zagg, zdeg):
    mesh = plsc.VectorSubcoreMesh(core_axis_name="c", subcore_axis_name="s")

    @functools.partial(
        pl.kernel,
        out_type=[
            jax.ShapeDtypeStruct((NC, B_PAD, D), jnp.float32),
            jax.ShapeDtypeStruct((NC, B_PAD, 16), jnp.float32),
        ],
        mesh=mesh,
        compiler_params=pltpu.CompilerParams(use_tc_tiling_on_sc=False),
        scratch_types=[
            pltpu.VMEM((CHUNKS_PER_W, CHUNK), jnp.int32),
            pltpu.VMEM((CHUNKS_PER_W, CHUNK), jnp.int32),
            pltpu.VMEM((CHUNK, D), jnp.float32),
            pltpu.VMEM((CHUNK, D), jnp.float32),
            pltpu.VMEM((CHUNK, 16), jnp.float32),
            pltpu.VMEM_SHARED((B_PAD, D), jnp.float32),
            pltpu.VMEM_SHARED((B_PAD, 16), jnp.float32),
            pltpu.SemaphoreType.DMA,
            pltpu.SemaphoreType.DMA,
            pltpu.SemaphoreType.DMA,
            pltpu.SemaphoreType.DMA,
            pltpu.SemaphoreType.DMA,
        ],
    )
    def sc_kernel(fused_hbm, iidx_hbm, bidx_hbm, ones_hbm, zagg_hbm, zdeg_hbm,
                  agg_out, deg_out, iidx_v, bidx_v, rows_a, rows_b, ones_v,
                  agg_s, deg_s, sga, sgb, ssa, ssb, so):
        c = lax.axis_index("c")
        s = lax.axis_index("s")
        w = s * NC + c
        pltpu.sync_copy(iidx_hbm.at[w], iidx_v)
        pltpu.sync_copy(bidx_hbm.at[w], bidx_v)
        pltpu.sync_copy(ones_hbm, ones_v)
        r0 = s * ROWS_PER_TILE
        pltpu.sync_copy(zagg_hbm.at[pl.ds(r0, ROWS_PER_TILE)],
                        agg_s.at[pl.ds(r0, ROWS_PER_TILE)])
        pltpu.sync_copy(zdeg_hbm.at[pl.ds(r0, ROWS_PER_TILE)],
                        deg_s.at[pl.ds(r0, ROWS_PER_TILE)])
        plsc.subcore_barrier()

        def gather(j, buf, sem):
            return pltpu.async_copy(fused_hbm.at[iidx_v.at[j]], buf, sem)

        def gather_wait(j, buf, sem):
            pltpu.make_async_copy(fused_hbm.at[iidx_v.at[j]], buf, sem).wait()

        def scatter(j, buf, sem):
            return pltpu.async_copy(buf, agg_s.at[bidx_v.at[j]], sem, add=True)

        def scatter_wait(j, buf, sem):
            pltpu.make_async_copy(buf, agg_s.at[bidx_v.at[j]], sem).wait()

        def ones_scatter(j):
            return pltpu.async_copy(ones_v, deg_s.at[bidx_v.at[j]], so, add=True)

        def ones_wait(j):
            pltpu.make_async_copy(ones_v, deg_s.at[bidx_v.at[j]], so).wait()

        half = CHUNKS_PER_W // 2
        gather(0, rows_a, sga)

        def body(t, carry):
            a = 2 * t
            b = a + 1
            gather_wait(a, rows_a, sga)

            @pl.when(t > 0)
            def _():
                scatter_wait(a, rows_b, ssb)
                ones_wait(a)
                ones_wait(a)

            gather(b, rows_b, sgb)
            scatter(a, rows_a, ssa)
            ones_scatter(a)
            gather_wait(b, rows_b, sgb)

            @pl.when(t < half - 1)
            def _():
                scatter_wait(a, rows_a, ssa)
                gather(a + 2, rows_a, sga)

            scatter(b, rows_b, ssb)
            ones_scatter(b)
            return carry

        lax.fori_loop(0, half, body, 0)
        scatter_wait(0, rows_a, ssa)
        scatter_wait(0, rows_b, ssb)
        ones_wait(0)
        ones_wait(0)
        plsc.subcore_barrier()
        pltpu.sync_copy(agg_s.at[pl.ds(r0, ROWS_PER_TILE)],
                        agg_out.at[c, pl.ds(r0, ROWS_PER_TILE)])
        pltpu.sync_copy(deg_s.at[pl.ds(r0, ROWS_PER_TILE)],
                        deg_out.at[c, pl.ds(r0, ROWS_PER_TILE)])

    return sc_kernel(fused, iidx, bidx, ones, zagg, zdeg)


def _combine_body(a_ref, d_ref, o_ref):
    a = a_ref[0] + a_ref[1]
    dd = d_ref[0][:, :1] + d_ref[1][:, :1]
    o_ref[...] = a / (dd + EPS)


def _combine_stage(agg_p, deg_p):
    return pl.pallas_call(
        _combine_body,
        grid=(B // CB_ROWS,),
        in_specs=[
            pl.BlockSpec((NC, CB_ROWS, D), lambda i: (0, i, 0)),
            pl.BlockSpec((NC, CB_ROWS, 16), lambda i: (0, i, 0)),
        ],
        out_specs=pl.BlockSpec((CB_ROWS, D), lambda i: (i, 0)),
        out_shape=jax.ShapeDtypeStruct((B, D), jnp.float32),
    )(agg_p, deg_p)


def kernel(content_feature, text_feature, cf_feature, item_idx, bundle_idx,
           c_W1, c_b1, c_W2, c_b2, c_W3, c_b3,
           t_W1, t_b1, t_W2, t_b2, t_W3, t_b3,
           cf_W, cf_b, item_emb, Wq, Wk, Wv):
    weights = [
        c_W1, c_b1.reshape(1, -1), c_W2, c_b2.reshape(1, -1),
        c_W3, c_b3.reshape(1, -1),
        t_W1, t_b1.reshape(1, -1), t_W2, t_b2.reshape(1, -1),
        t_W3, t_b3.reshape(1, -1),
        cf_W, cf_b.reshape(1, -1), Wq, Wk, Wv,
    ]
    fused = _dense_stage(content_feature, text_feature, cf_feature, item_emb,
                         weights)

    pad = E_PAD - E
    iidx = jnp.concatenate(
        [item_idx.astype(jnp.int32), jnp.zeros((pad,), jnp.int32)]
    ).reshape(NW, CHUNKS_PER_W, CHUNK)
    bidx = jnp.concatenate(
        [bundle_idx.astype(jnp.int32), jnp.full((pad,), B_PAD - 1, jnp.int32)]
    ).reshape(NW, CHUNKS_PER_W, CHUNK)
    ones = jnp.ones((CHUNK, 16), jnp.float32)
    zagg = jnp.zeros((B_PAD, D), jnp.float32)
    zdeg = jnp.zeros((B_PAD, 16), jnp.float32)

    agg_p, deg_p = _sc_stage(fused, iidx, bidx, ones, zagg, zdeg)
    return _combine_stage(agg_p, deg_p)


# speedup vs baseline: 7.7441x; 1.4125x over previous
import functools

import jax
import jax.numpy as jnp
from jax import lax
from jax.experimental import pallas as pl
from jax.experimental.pallas import tpu as pltpu
from jax.experimental.pallas import tpu_sc as plsc

N = 10000
B = 5000
E = 160000
D = 64
EPS = 1e-9

NC, NS = 2, 16
NW = NC * NS
CHUNK = 128
CHUNKS_PER_W = 40
E_PAD = NW * CHUNKS_PER_W * CHUNK
B_PAD = 5120
ROWS_PER_TILE = B_PAD // NS

ROW_BLOCK = 1000
CB_ROWS = 1000


def _dense_body(cf_ref, tf_ref, cff_ref, ie_ref,
                cW1, cb1, cW2, cb2, cW3, cb3,
                tW1, tb1, tW2, tb2, tW3, tb3,
                cfW, cfb, Wq, Wk, Wv, out_ref):
    def norm_rows(x):
        n = jnp.sqrt(jnp.sum(x * x, axis=-1, keepdims=True))
        return x / jnp.maximum(n, EPS)

    def mlp(x, W1, b1, W2, b2, W3, b3):
        h = jnp.maximum(
            jnp.dot(x, W1[...], preferred_element_type=jnp.float32) + b1[...], 0.0)
        h = jnp.maximum(
            jnp.dot(h, W2[...], preferred_element_type=jnp.float32) + b2[...], 0.0)
        return jnp.dot(h, W3[...], preferred_element_type=jnp.float32) + b3[...]

    c = mlp(norm_rows(cf_ref[...]), cW1, cb1, cW2, cb2, cW3, cb3)
    t = mlp(norm_rows(tf_ref[...]), tW1, tb1, tW2, tb2, tW3, tb3)
    cfl = jnp.dot(cff_ref[...], cfW[...], preferred_element_type=jnp.float32) + cfb[...]
    feats = [c, t, cfl, ie_ref[...]]
    q = [jnp.dot(f, Wq[...], preferred_element_type=jnp.float32) for f in feats]
    k = [jnp.dot(f, Wk[...], preferred_element_type=jnp.float32) for f in feats]
    v = [jnp.dot(f, Wv[...], preferred_element_type=jnp.float32) for f in feats]
    acc = None
    for m in range(4):
        s = [jnp.sum(q[m] * k[j], axis=-1, keepdims=True) * 0.125 for j in range(4)]
        mx = jnp.maximum(jnp.maximum(s[0], s[1]), jnp.maximum(s[2], s[3]))
        e = [jnp.exp(sj - mx) for sj in s]
        den = e[0] + e[1] + e[2] + e[3]
        o = (e[0] * v[0] + e[1] * v[1] + e[2] * v[2] + e[3] * v[3]) / den
        mu = jnp.mean(o, axis=-1, keepdims=True)
        var = jnp.mean((o - mu) ** 2, axis=-1, keepdims=True)
        o = (o - mu) * lax.rsqrt(var + 1e-5)
        acc = o if acc is None else acc + o
    out_ref[...] = acc * 0.25


def _dense_stage(content, text, cff, ie, weights):
    row = lambda i: (i, 0)
    full = lambda i: (0, 0)
    in_specs = [
        pl.BlockSpec((ROW_BLOCK, 256), row),
        pl.BlockSpec((ROW_BLOCK, 256), row),
        pl.BlockSpec((ROW_BLOCK, D), row),
        pl.BlockSpec((ROW_BLOCK, D), row),
    ] + [pl.BlockSpec(w.shape, full) for w in weights]
    return pl.pallas_call(
        _dense_body,
        grid=(N // ROW_BLOCK,),
        in_specs=in_specs,
        out_specs=pl.BlockSpec((ROW_BLOCK, D), row),
        out_shape=jax.ShapeDtypeStruct((N, D), jnp.float32),
    )(content, text, cff, ie, *weights)


def _sc_stage(fused, iidx, bidx, ones, zagg, zdeg):
    mesh = plsc.VectorSubcoreMesh(core_axis_name="c", subcore_axis_name="s")

    @functools.partial(
        pl.kernel,
        out_type=[
            jax.ShapeDtypeStruct((NC, B_PAD, D), jnp.float32),
            jax.ShapeDtypeStruct((NC, B_PAD, 16), jnp.float32),
        ],
        mesh=mesh,
        compiler_params=pltpu.CompilerParams(use_tc_tiling_on_sc=False),
        scratch_types=[
            pltpu.VMEM((CHUNKS_PER_W, CHUNK), jnp.int32),
            pltpu.VMEM((CHUNKS_PER_W, CHUNK), jnp.int32),
            pltpu.VMEM((CHUNK, D), jnp.float32),
            pltpu.VMEM((CHUNK, D), jnp.float32),
            pltpu.VMEM((CHUNK, 16), jnp.float32),
            pltpu.VMEM_SHARED((B_PAD, D), jnp.float32),
            pltpu.VMEM_SHARED((B_PAD, 16), jnp.float32),
            pltpu.VMEM_SHARED((N, D), jnp.float32),
            pltpu.SemaphoreType.DMA,
            pltpu.SemaphoreType.DMA,
            pltpu.SemaphoreType.DMA,
            pltpu.SemaphoreType.DMA,
            pltpu.SemaphoreType.DMA,
        ],
    )
    def sc_kernel(fused_hbm, iidx_hbm, bidx_hbm, ones_hbm, zagg_hbm, zdeg_hbm,
                  agg_out, deg_out, iidx_v, bidx_v, rows_a, rows_b, ones_v,
                  agg_s, deg_s, table_s, sga, sgb, ssa, ssb, so):
        c = lax.axis_index("c")
        s = lax.axis_index("s")
        w = s * NC + c
        pltpu.sync_copy(iidx_hbm.at[w], iidx_v)
        pltpu.sync_copy(bidx_hbm.at[w], bidx_v)
        pltpu.sync_copy(ones_hbm, ones_v)
        r0 = s * ROWS_PER_TILE
        pltpu.sync_copy(zagg_hbm.at[pl.ds(r0, ROWS_PER_TILE)],
                        agg_s.at[pl.ds(r0, ROWS_PER_TILE)])
        pltpu.sync_copy(zdeg_hbm.at[pl.ds(r0, ROWS_PER_TILE)],
                        deg_s.at[pl.ds(r0, ROWS_PER_TILE)])
        t0 = s * (N // NS)
        pltpu.sync_copy(fused_hbm.at[pl.ds(t0, N // NS)],
                        table_s.at[pl.ds(t0, N // NS)])
        plsc.subcore_barrier()

        def gather(j, buf, sem):
            return pltpu.async_copy(table_s.at[iidx_v.at[j]], buf, sem)

        def gather_wait(j, buf, sem):
            pltpu.make_async_copy(table_s.at[iidx_v.at[j]], buf, sem).wait()

        def scatter(j, buf, sem):
            return pltpu.async_copy(buf, agg_s.at[bidx_v.at[j]], sem, add=True)

        def scatter_wait(j, buf, sem):
            pltpu.make_async_copy(buf, agg_s.at[bidx_v.at[j]], sem).wait()

        def ones_scatter(j):
            return pltpu.async_copy(ones_v, deg_s.at[bidx_v.at[j]], so, add=True)

        def ones_wait(j):
            pltpu.make_async_copy(ones_v, deg_s.at[bidx_v.at[j]], so).wait()

        half = CHUNKS_PER_W // 2
        gather(0, rows_a, sga)

        def body(t, carry):
            a = 2 * t
            b = a + 1
            gather_wait(a, rows_a, sga)

            @pl.when(t > 0)
            def _():
                scatter_wait(a, rows_b, ssb)
                ones_wait(a)
                ones_wait(a)

            gather(b, rows_b, sgb)
            scatter(a, rows_a, ssa)
            ones_scatter(a)
            gather_wait(b, rows_b, sgb)

            @pl.when(t < half - 1)
            def _():
                scatter_wait(a, rows_a, ssa)
                gather(a + 2, rows_a, sga)

            scatter(b, rows_b, ssb)
            ones_scatter(b)
            return carry

        lax.fori_loop(0, half, body, 0)
        scatter_wait(0, rows_a, ssa)
        scatter_wait(0, rows_b, ssb)
        ones_wait(0)
        ones_wait(0)
        plsc.subcore_barrier()
        pltpu.sync_copy(agg_s.at[pl.ds(r0, ROWS_PER_TILE)],
                        agg_out.at[c, pl.ds(r0, ROWS_PER_TILE)])
        pltpu.sync_copy(deg_s.at[pl.ds(r0, ROWS_PER_TILE)],
                        deg_out.at[c, pl.ds(r0, ROWS_PER_TILE)])

    return sc_kernel(fused, iidx, bidx, ones, zagg, zdeg)


def _combine_body(a_ref, d_ref, o_ref):
    a = a_ref[0] + a_ref[1]
    dd = d_ref[0][:, :1] + d_ref[1][:, :1]
    o_ref[...] = a / (dd + EPS)


def _combine_stage(agg_p, deg_p):
    return pl.pallas_call(
        _combine_body,
        grid=(B // CB_ROWS,),
        in_specs=[
            pl.BlockSpec((NC, CB_ROWS, D), lambda i: (0, i, 0)),
            pl.BlockSpec((NC, CB_ROWS, 16), lambda i: (0, i, 0)),
        ],
        out_specs=pl.BlockSpec((CB_ROWS, D), lambda i: (i, 0)),
        out_shape=jax.ShapeDtypeStruct((B, D), jnp.float32),
    )(agg_p, deg_p)


def kernel(content_feature, text_feature, cf_feature, item_idx, bundle_idx,
           c_W1, c_b1, c_W2, c_b2, c_W3, c_b3,
           t_W1, t_b1, t_W2, t_b2, t_W3, t_b3,
           cf_W, cf_b, item_emb, Wq, Wk, Wv):
    weights = [
        c_W1, c_b1.reshape(1, -1), c_W2, c_b2.reshape(1, -1),
        c_W3, c_b3.reshape(1, -1),
        t_W1, t_b1.reshape(1, -1), t_W2, t_b2.reshape(1, -1),
        t_W3, t_b3.reshape(1, -1),
        cf_W, cf_b.reshape(1, -1), Wq, Wk, Wv,
    ]
    fused = _dense_stage(content_feature, text_feature, cf_feature, item_emb,
                         weights)

    pad = E_PAD - E
    iidx = jnp.concatenate(
        [item_idx.astype(jnp.int32), jnp.zeros((pad,), jnp.int32)]
    ).reshape(NW, CHUNKS_PER_W, CHUNK)
    bidx = jnp.concatenate(
        [bundle_idx.astype(jnp.int32), jnp.full((pad,), B_PAD - 1, jnp.int32)]
    ).reshape(NW, CHUNKS_PER_W, CHUNK)
    ones = jnp.ones((CHUNK, 16), jnp.float32)
    zagg = jnp.zeros((B_PAD, D), jnp.float32)
    zdeg = jnp.zeros((B_PAD, 16), jnp.float32)

    agg_p, deg_p = _sc_stage(fused, iidx, bidx, ones, zagg, zdeg)
    return _combine_stage(agg_p, deg_p)
